# baseline (speedup 1.0000x reference)
import functools

import jax
import jax.numpy as jnp
from jax import lax
from jax.experimental import pallas as pl
from jax.experimental.pallas import tpu as pltpu
from jax.experimental.pallas import tpu_sc as plsc

B = 16
N = 2048
S = 512
CIN = 64
EPS = 1e-5
RADII = (0.1, 0.2, 0.4)
KS = (16, 32, 128)
DP = 128
NW = 32
SB = S // 2
_CH = 128


def _fps_body(x_ref, y_ref, z_ref, f_ref, ox_ref, oy_ref, oz_ref, dist_ref):
    x = x_ref[...]
    y = y_ref[...]
    z = z_ref[...]
    iota = lax.broadcasted_iota(jnp.int32, (B, N), 1)
    iota_s = lax.broadcasted_iota(jnp.int32, (B, S), 1)
    dist_ref[...] = jnp.full((B, N), 1e10, jnp.float32)
    ox_ref[...] = jnp.zeros((B, S), jnp.float32)
    oy_ref[...] = jnp.zeros((B, S), jnp.float32)
    oz_ref[...] = jnp.zeros((B, S), jnp.float32)

    def body(s, nidx):
        mf = (iota == nidx).astype(jnp.float32)
        cx = jnp.sum(x * mf, axis=1, keepdims=True)
        cy = jnp.sum(y * mf, axis=1, keepdims=True)
        cz = jnp.sum(z * mf, axis=1, keepdims=True)
        sel = iota_s == s
        ox_ref[...] = jnp.where(sel, cx, ox_ref[...])
        oy_ref[...] = jnp.where(sel, cy, oy_ref[...])
        oz_ref[...] = jnp.where(sel, cz, oz_ref[...])
        dx = x - cx
        dy = y - cy
        dz = z - cz
        d = dx * dx + dy * dy
        d = d + dz * dz
        dist = jnp.minimum(dist_ref[...], d)
        dist_ref[...] = dist
        mx = jnp.max(dist, axis=1, keepdims=True)
        return jnp.min(jnp.where(dist == mx, iota, N), axis=1, keepdims=True)

    lax.fori_loop(0, S, body, f_ref[...])


def _fps(x, y, z, far0):
    out = jax.ShapeDtypeStruct((B, S), jnp.float32)
    return pl.pallas_call(
        _fps_body,
        out_shape=(out, out, out),
        scratch_shapes=[pltpu.VMEM((B, N), jnp.float32)],
    )(x, y, z, far0)


_SBD = 128


def _dist_rank_body(specs, c_ref, p_ref, pos_ref, cnt_ref):
    a = c_ref[0]
    pb = p_ref[0]
    mm = jnp.dot(a, pb)
    ssrc = jnp.sum(a * a, axis=1, keepdims=True)
    sdst = jnp.sum(pb * pb, axis=0, keepdims=True)
    d = -2.0 * mm
    d = d + ssrc
    d = d + sdst
    ii = lax.broadcasted_iota(jnp.int32, (_CH, _CH), 0)
    jj = lax.broadcasted_iota(jnp.int32, (_CH, _CH), 1)
    tri = (ii <= jj).astype(jnp.float32)
    lane = lax.broadcasted_iota(jnp.int32, (_SBD, _CH), 1)
    bases = [jnp.zeros((_SBD, 1), jnp.float32) for _ in specs]
    for c in range(N // _CH):
        dc = d[:, c * _CH:(c + 1) * _CH]
        pk = jnp.zeros((_SBD, _CH), jnp.int32)
        for q, (r2, kq) in enumerate(specs):
            mc = (dc <= r2).astype(jnp.float32)
            pref = jnp.dot(mc, tri)
            rank = pref + bases[q] - 1.0
            valid = (mc > 0.0) & (rank < kq)
            dump = jnp.asarray(float(kq), jnp.float32) + (lane % 16).astype(
                jnp.float32)
            pq = jnp.where(valid, rank, dump).astype(jnp.int32)
            pk = pk + (pq << (8 * q))
            last = jnp.sum(jnp.where(lane == _CH - 1, pref, 0.0),
                           axis=1, keepdims=True)
            bases[q] = bases[q] + last
        pos_ref[0, :, c * _CH:(c + 1) * _CH] = pk
    cnt = jnp.zeros((_SBD, 1), jnp.int32)
    for q, (r2, kq) in enumerate(specs):
        cq = jnp.minimum(bases[q], float(kq)).astype(jnp.int32)
        cnt = cnt + (cq << (8 * q))
    cnt_ref[0, :, :] = jnp.broadcast_to(cnt, (_SBD, 8))


def _dist_rank(cen4, p4, specs):
    return pl.pallas_call(
        functools.partial(_dist_rank_body, specs),
        grid=(B, S // _SBD),
        in_specs=[
            pl.BlockSpec((1, _SBD, 4), lambda b, i: (b, i, 0)),
            pl.BlockSpec((1, 4, N), lambda b, i: (b, 0, 0)),
        ],
        out_specs=[
            pl.BlockSpec((1, _SBD, N), lambda b, i: (b, i, 0)),
            pl.BlockSpec((1, _SBD, 8), lambda b, i: (b, i, 0)),
        ],
        out_shape=[
            jax.ShapeDtypeStruct((B, S, N), jnp.int32),
            jax.ShapeDtypeStruct((B, S, 8), jnp.int32),
        ],
    )(cen4, p4)


def _make_ballq(specs):
    nq = len(specs)
    ks = [k for _, k in specs]
    sumk = sum(ks)
    koff = [sum(ks[:q]) for q in range(nq)]
    mesh = plsc.VectorSubcoreMesh(core_axis_name="c", subcore_axis_name="s")
    scratch = [pltpu.VMEM((N,), jnp.int32), pltpu.VMEM((N,), jnp.int32),
               pltpu.VMEM((SB, 8), jnp.int32), pltpu.VMEM((sumk,), jnp.int32)]
    scratch += [pltpu.VMEM((k + 16,), jnp.int32) for k in ks]
    scratch += [pltpu.SemaphoreType.DMA, pltpu.SemaphoreType.DMA]

    @functools.partial(
        pl.kernel,
        out_type=jax.ShapeDtypeStruct((B, S, sumk), jnp.int32),
        mesh=mesh,
        scratch_types=scratch,
        compiler_params=pltpu.CompilerParams(needs_layout_passes=False),
    )
    def k(pos_hbm, cnt_hbm, out, prow0, prow1, cnt_v, obuf, *rest):
        rbufs = rest[:nq]
        sin0, sin1 = rest[nq], rest[nq + 1]
        wid = lax.axis_index("s") * 2 + lax.axis_index("c")
        b = wid // 2
        s0 = (wid % 2) * SB
        bn = b * N
        l16 = lax.broadcasted_iota(jnp.int32, (16,), 0)
        z16 = jnp.zeros((16,), jnp.int32)
        pltpu.sync_copy(cnt_hbm.at[b, pl.ds(s0, SB)], cnt_v)
        pltpu.async_copy(pos_hbm.at[b, s0], prow0, sin0)
        pltpu.async_copy(pos_hbm.at[b, s0 + 1], prow1, sin1)

        def process(i, prow):
            for q in range(nq):
                rbufs[q][pl.ds(0, 16)] = z16 + bn

            def vec_body(u, _c):
                for w in range(4):
                    v = u * 4 + w
                    pv = prow[pl.ds(v * 16, 16)]
                    idxv = l16 + (bn + v * 16)
                    for q in range(nq):
                        pe = (pv >> (8 * q)) & 255
                        plsc.store_scatter(rbufs[q], [pe], idxv)
                return 0

            lax.fori_loop(0, N // 64, vec_body, 0)
            iv = z16 + i
            for q in range(nq):
                cql = plsc.load_gather(cnt_v, [iv, z16])
                cq = (cql >> (8 * q)) & 255
                for j in range(ks[q] // 16):
                    posj = l16 + j * 16
                    gidx = jnp.where(posj < cq, posj, 0)
                    vec = plsc.load_gather(rbufs[q], [gidx])
                    obuf[pl.ds(koff[q] + j * 16, 16)] = vec
            pltpu.sync_copy(obuf, out.at[b, s0 + i])

        def row_body(j, _):
            i0 = 2 * j
            pltpu.make_async_copy(pos_hbm.at[b, s0], prow0, sin0).wait()
            process(i0, prow0)
            pltpu.async_copy(
                pos_hbm.at[b, s0 + jnp.minimum(i0 + 2, SB - 1)], prow0, sin0)
            pltpu.make_async_copy(pos_hbm.at[b, s0], prow1, sin1).wait()
            process(i0 + 1, prow1)
            pltpu.async_copy(
                pos_hbm.at[b, s0 + jnp.minimum(i0 + 3, SB - 1)], prow1, sin1)
            return 0

        lax.fori_loop(0, SB // 2, row_body, 0)
        pltpu.make_async_copy(pos_hbm.at[b, s0], prow0, sin0).wait()
        pltpu.make_async_copy(pos_hbm.at[b, s0], prow1, sin1).wait()

    return k


def _make_gather(rows, dtype=jnp.float32):
    per_w = rows // NW
    ch = 256 if per_w % 256 == 0 else 128
    n_ch = per_w // ch
    assert n_ch % 2 == 0 or n_ch == 1
    mesh = plsc.VectorSubcoreMesh(core_axis_name="c", subcore_axis_name="s")

    @functools.partial(
        pl.kernel,
        out_type=jax.ShapeDtypeStruct((rows, DP), dtype),
        mesh=mesh,
        scratch_types=[
            pltpu.VMEM((ch,), jnp.int32), pltpu.VMEM((ch,), jnp.int32),
            pltpu.VMEM((ch, DP), dtype), pltpu.VMEM((ch, DP), dtype),
            pltpu.SemaphoreType.DMA, pltpu.SemaphoreType.DMA,
            pltpu.SemaphoreType.DMA, pltpu.SemaphoreType.DMA,
        ],
    )
    def k(tab_hbm, idx_hbm, out_hbm, idx_a, idx_b, rows_a, rows_b,
          sga, sgb, soa, sob):
        wid = lax.axis_index("s") * 2 + lax.axis_index("c")
        base_w = wid * per_w

        def fire(c, idx_v, rows_v, sem):
            pltpu.sync_copy(idx_hbm.at[pl.ds(base_w + c * ch, ch)], idx_v)
            for j in range(ch // 128):
                pltpu.async_copy(
                    tab_hbm.at[idx_v.at[pl.ds(j * 128, 128)]],
                    rows_v.at[pl.ds(j * 128, 128)], sem)

        def gwait(idx_v, rows_v, sem):
            for j in range(ch // 128):
                pltpu.make_async_copy(
                    tab_hbm.at[idx_v.at[pl.ds(j * 128, 128)]],
                    rows_v.at[pl.ds(j * 128, 128)], sem).wait()

        def owait(rows_v, c, sem):
            pltpu.make_async_copy(
                rows_v, out_hbm.at[pl.ds(base_w, ch)], sem).wait()

        if n_ch == 1:
            fire(0, idx_a, rows_a, sga)
            gwait(idx_a, rows_a, sga)
            pltpu.sync_copy(rows_a, out_hbm.at[pl.ds(base_w, ch)])
            return

        fire(0, idx_a, rows_a, sga)
        fire(1, idx_b, rows_b, sgb)

        def body(h, _):
            c0 = 2 * h
            gwait(idx_a, rows_a, sga)
            pltpu.async_copy(rows_a, out_hbm.at[pl.ds(base_w + c0 * ch, ch)],
                             soa)
            gwait(idx_b, rows_b, sgb)
            pltpu.async_copy(
                rows_b, out_hbm.at[pl.ds(base_w + (c0 + 1) * ch, ch)], sob)
            nx0 = jnp.minimum(c0 + 2, n_ch - 2)
            nx1 = jnp.minimum(c0 + 3, n_ch - 1)
            owait(rows_a, c0, soa)
            fire(nx0, idx_a, rows_a, sga)
            owait(rows_b, c0 + 1, sob)
            fire(nx1, idx_b, rows_b, sgb)
            return 0

        lax.fori_loop(0, n_ch // 2, body, 0)
        gwait(idx_a, rows_a, sga)
        gwait(idx_b, rows_b, sgb)

    return k


def _prep_layers(params, pad_first_to=DP):
    out = []
    for i, (W, b, gamma, beta) in enumerate(params):
        Wt = jnp.transpose(W)
        if i == 0 and Wt.shape[0] < pad_first_to:
            Wt = jnp.concatenate(
                [Wt, jnp.zeros((pad_first_to - Wt.shape[0], Wt.shape[1]),
                               jnp.float32)], axis=0)
        s = gamma / jnp.sqrt(1.0 + EPS)
        out.append((Wt, b[None, :], s[None, :], beta[None, :]))
    return out


def _mlp(x, wrefs):
    for (w, b, s, t) in wrefs:
        y = jnp.dot(x, w[...])
        y = (y + b[...]) * s[...] + t[...]
        x = jnp.maximum(y, 0.0)
    return x


def _adj_center(x3, cen, cpb, kk):
    li4 = lax.broadcasted_iota(jnp.int32, (cpb, 4), 1)
    li = lax.broadcasted_iota(jnp.int32, (cpb, kk, DP), 2)
    adj = jnp.zeros((cpb, kk, DP), jnp.float32)
    for d in range(3):
        cd = jnp.sum(jnp.where(li4 == d, cen, 0.0), axis=1, keepdims=True)
        adj = adj + jnp.where(li == 64 + d, cd[:, :, None], 0.0)
    return x3 - adj


_RCPB = 64


def _refine_body(nl, g_ref, c_ref, *rest):
    wrefs = [tuple(rest[4 * i:4 * i + 4]) for i in range(nl)]
    out_ref = rest[4 * nl]
    x3 = g_ref[...]
    cen = c_ref[...]
    xf = _adj_center(x3, cen, _RCPB, 16)
    x2 = xf.reshape(_RCPB * 16, DP)
    h = _mlp(x2, wrefs)
    h3 = h.reshape(_RCPB, 16, 1)
    m = h3
    for hh in (8, 4, 2, 1):
        m = jnp.maximum(m[:, :hh], m[:, hh:2 * hh])
    e = jnp.exp(h3 - m)
    ssum = e
    for hh in (8, 4, 2, 1):
        ssum = ssum[:, :hh] + ssum[:, hh:2 * hh]
    p = e / ssum
    w = p * x3
    for hh in (8, 4, 2, 1):
        w = w[:, :hh] + w[:, hh:2 * hh]
    out_ref[...] = w.reshape(_RCPB, DP)


def _refine(g0, cen0, layers):
    nl = len(layers)
    flat = [a for l in layers for a in l]
    wspecs = [pl.BlockSpec(a.shape, lambda i: tuple(0 for _ in a.shape))
              for a in flat]
    return pl.pallas_call(
        functools.partial(_refine_body, nl),
        grid=(B * S // _RCPB,),
        in_specs=[
            pl.BlockSpec((_RCPB, 16, DP), lambda i: (i, 0, 0)),
            pl.BlockSpec((_RCPB, 4), lambda i: (i, 0)),
        ] + wspecs,
        out_specs=pl.BlockSpec((_RCPB, DP), lambda i: (i, 0)),
        out_shape=jax.ShapeDtypeStruct((B * S, DP), jnp.float32),
    )(g0, cen0, *flat)


def _branch_body(nl, kk, cpb, g_ref, c_ref, *rest):
    wrefs = [tuple(rest[4 * i:4 * i + 4]) for i in range(nl)]
    out_ref = rest[4 * nl]
    x3 = g_ref[...]
    cen = c_ref[...]
    xf = _adj_center(x3, cen, cpb, kk)
    x2 = xf.reshape(cpb * kk, DP)
    y = _mlp(x2, wrefs)
    c_out = y.shape[1]
    y3 = y.reshape(cpb, kk, c_out)
    hh = kk // 2
    while hh >= 1:
        y3 = jnp.maximum(y3[:, :hh], y3[:, hh:2 * hh])
        hh //= 2
    out_ref[...] = y3.reshape(cpb, c_out)


def _branch(g, cen, layers, kk):
    nl = len(layers)
    c_out = layers[-1][0].shape[1]
    cpb = max(1, 1024 // kk)
    flat = [a for l in layers for a in l]
    wspecs = [pl.BlockSpec(a.shape, lambda i: tuple(0 for _ in a.shape))
              for a in flat]
    return pl.pallas_call(
        functools.partial(_branch_body, nl, kk, cpb),
        grid=(B * S // cpb,),
        in_specs=[
            pl.BlockSpec((cpb, kk, DP), lambda i: (i, 0, 0)),
            pl.BlockSpec((cpb, 4), lambda i: (i, 0)),
        ] + wspecs,
        out_specs=pl.BlockSpec((cpb, c_out), lambda i: (i, 0)),
        out_shape=jax.ShapeDtypeStruct((B * S, c_out), jnp.float32),
    )(g, cen, *flat)


def _ballq_jax_TEMP(specs):
    def f(pos, cnt):
        outs = []
        for q, (r2, kq) in enumerate(specs):
            pq = (pos >> (8 * q)) & 255
            vals = jnp.where(pq < kq, pq, 255)
            idxs = jnp.argsort(vals, axis=-1, stable=True
                               ).astype(jnp.int32)[:, :, :kq]
            cq = (cnt[:, :, 0] >> (8 * q)) & 255
            kpos = jnp.arange(kq, dtype=jnp.int32)[None, None, :]
            first = idxs[:, :, :1]
            out = jnp.where(kpos < cq[:, :, None], idxs, first)
            out = jnp.where(cq[:, :, None] == 0, 0, out)
            out = out + jnp.arange(B, dtype=jnp.int32)[:, None, None] * N
            outs.append(out)
        return tuple(outs)
    return f


def kernel(xyz, points, refine_params, msg_params):
    x = xyz[:, 0, :]
    y = xyz[:, 1, :]
    z = xyz[:, 2, :]
    far0 = jax.random.randint(jax.random.key(1), (B,), 0, N,
                              dtype=jnp.int32).reshape(B, 1)
    ox, oy, oz = _fps(x, y, z, far0)
    cen0 = jnp.stack([ox, oy, oz, jnp.zeros_like(ox)], axis=-1)
    p4 = jnp.concatenate([xyz, jnp.zeros((B, 1, N), jnp.float32)], axis=1)
    spec0 = ((RADII[0] ** 2, 16),)
    pos0, cnt0 = _dist_rank(cen0, p4, spec0)
    g0 = _make_ballq(spec0)(pos0, cnt0)

    tab = jnp.concatenate(
        [jnp.transpose(points, (0, 2, 1)), jnp.transpose(xyz, (0, 2, 1)),
         jnp.zeros((B, N, DP - CIN - 3), jnp.float32)],
        axis=-1).reshape(B * N, DP)

    G0 = _make_gather(B * S * 16)(tab, g0.reshape(-1))
    rlayers = _prep_layers(refine_params)
    cen1_dp = _refine(G0.reshape(B * S, 16, DP), cen0.reshape(B * S, 4),
                      rlayers)
    cen1 = cen1_dp[:, 64:67]
    cen1_4 = jnp.concatenate(
        [cen1, jnp.zeros((B * S, 1), jnp.float32)], axis=1)

    spec1 = tuple((r ** 2, k) for r, k in zip(RADII, KS))
    pos1, cnt1 = _dist_rank(cen1_4.reshape(B, S, 4), p4, spec1)
    gall = _make_ballq(spec1)(pos1, cnt1)
    gs = (gall[:, :, :16], gall[:, :, 16:48], gall[:, :, 48:176])

    outs = []
    for i, kk in enumerate(KS):
        Gi = _make_gather(B * S * kk)(tab, gs[i].reshape(-1))
        blayers = _prep_layers(msg_params[i])
        outs.append(_branch(Gi.reshape(B * S, kk, DP), cen1_4, blayers, kk))

    new_xyz_out = jnp.transpose(cen1.reshape(B, S, 3), (0, 2, 1))
    new_points = jnp.concatenate(
        [jnp.transpose(o.reshape(B, S, -1), (0, 2, 1)) for o in outs], axis=1)
    return new_xyz_out, new_points


# speedup vs baseline: 1.0004x; 1.0004x over previous
import functools

import jax
import jax.numpy as jnp
from jax import lax
from jax.experimental import pallas as pl
from jax.experimental.pallas import tpu as pltpu
from jax.experimental.pallas import tpu_sc as plsc

B = 16
N = 2048
S = 512
CIN = 64
EPS = 1e-5
RADII = (0.1, 0.2, 0.4)
KS = (16, 32, 128)
DP = 128
NW = 32
SB = S // 2
_CH = 128


def _fps_body(x_ref, y_ref, z_ref, f_ref, ox_ref, oy_ref, oz_ref, dist_ref):
    x = x_ref[...]
    y = y_ref[...]
    z = z_ref[...]
    iota = lax.broadcasted_iota(jnp.int32, (B, N), 1)
    iota_s = lax.broadcasted_iota(jnp.int32, (B, S), 1)
    dist_ref[...] = jnp.full((B, N), 1e10, jnp.float32)
    ox_ref[...] = jnp.zeros((B, S), jnp.float32)
    oy_ref[...] = jnp.zeros((B, S), jnp.float32)
    oz_ref[...] = jnp.zeros((B, S), jnp.float32)

    def body(s, nidx):
        mf = (iota == nidx).astype(jnp.float32)
        cx = jnp.sum(x * mf, axis=1, keepdims=True)
        cy = jnp.sum(y * mf, axis=1, keepdims=True)
        cz = jnp.sum(z * mf, axis=1, keepdims=True)
        sel = iota_s == s
        ox_ref[...] = jnp.where(sel, cx, ox_ref[...])
        oy_ref[...] = jnp.where(sel, cy, oy_ref[...])
        oz_ref[...] = jnp.where(sel, cz, oz_ref[...])
        dx = x - cx
        dy = y - cy
        dz = z - cz
        d = dx * dx + dy * dy
        d = d + dz * dz
        dist = jnp.minimum(dist_ref[...], d)
        dist_ref[...] = dist
        mx = jnp.max(dist, axis=1, keepdims=True)
        return jnp.min(jnp.where(dist == mx, iota, N), axis=1, keepdims=True)

    lax.fori_loop(0, S, body, f_ref[...])


def _fps(x, y, z, far0):
    out = jax.ShapeDtypeStruct((B, S), jnp.float32)
    return pl.pallas_call(
        _fps_body,
        out_shape=(out, out, out),
        scratch_shapes=[pltpu.VMEM((B, N), jnp.float32)],
    )(x, y, z, far0)


_SBD = 128


def _dist_rank_body(specs, c_ref, p_ref, pos_ref, cnt_ref):
    a = c_ref[0]
    pb = p_ref[0]
    mm = jnp.dot(a, pb)
    ssrc = jnp.sum(a * a, axis=1, keepdims=True)
    sdst = jnp.sum(pb * pb, axis=0, keepdims=True)
    d = -2.0 * mm
    d = d + ssrc
    d = d + sdst
    ii = lax.broadcasted_iota(jnp.int32, (_CH, _CH), 0)
    jj = lax.broadcasted_iota(jnp.int32, (_CH, _CH), 1)
    tri = (ii <= jj).astype(jnp.float32)
    lane = lax.broadcasted_iota(jnp.int32, (_SBD, _CH), 1)
    bases = [jnp.zeros((_SBD, 1), jnp.float32) for _ in specs]
    for c in range(N // _CH):
        dc = d[:, c * _CH:(c + 1) * _CH]
        pk = jnp.zeros((_SBD, _CH), jnp.int32)
        for q, (r2, kq) in enumerate(specs):
            mc = (dc <= r2).astype(jnp.float32)
            pref = jnp.dot(mc, tri)
            rank = pref + bases[q] - 1.0
            valid = (mc > 0.0) & (rank < kq)
            dump = jnp.asarray(float(kq), jnp.float32) + (lane % 16).astype(
                jnp.float32)
            pq = jnp.where(valid, rank, dump).astype(jnp.int32)
            pk = pk + (pq << (8 * q))
            last = jnp.sum(jnp.where(lane == _CH - 1, pref, 0.0),
                           axis=1, keepdims=True)
            bases[q] = bases[q] + last
        pos_ref[0, :, c * _CH:(c + 1) * _CH] = pk
    cnt = jnp.zeros((_SBD, 1), jnp.int32)
    for q, (r2, kq) in enumerate(specs):
        cq = jnp.minimum(bases[q], float(kq)).astype(jnp.int32)
        cnt = cnt + (cq << (8 * q))
    cnt_ref[0, :, :] = jnp.broadcast_to(cnt, (_SBD, 8))


def _dist_rank(cen4, p4, specs):
    return pl.pallas_call(
        functools.partial(_dist_rank_body, specs),
        grid=(B, S // _SBD),
        in_specs=[
            pl.BlockSpec((1, _SBD, 4), lambda b, i: (b, i, 0)),
            pl.BlockSpec((1, 4, N), lambda b, i: (b, 0, 0)),
        ],
        out_specs=[
            pl.BlockSpec((1, _SBD, N), lambda b, i: (b, i, 0)),
            pl.BlockSpec((1, _SBD, 8), lambda b, i: (b, i, 0)),
        ],
        out_shape=[
            jax.ShapeDtypeStruct((B, S, N), jnp.int32),
            jax.ShapeDtypeStruct((B, S, 8), jnp.int32),
        ],
    )(cen4, p4)


def _make_ballq(specs):
    nq = len(specs)
    ks = [k for _, k in specs]
    sumk = sum(ks)
    koff = [sum(ks[:q]) for q in range(nq)]
    mesh = plsc.VectorSubcoreMesh(core_axis_name="c", subcore_axis_name="s")
    scratch = [pltpu.VMEM((N,), jnp.int32) for _ in range(4)]
    scratch += [pltpu.VMEM((SB, 8), jnp.int32), pltpu.VMEM((sumk,), jnp.int32)]
    scratch += [pltpu.VMEM((k + 16,), jnp.int32) for k in ks]
    scratch += [pltpu.SemaphoreType.DMA for _ in range(4)]

    @functools.partial(
        pl.kernel,
        out_type=jax.ShapeDtypeStruct((B, S, sumk), jnp.int32),
        mesh=mesh,
        scratch_types=scratch,
        compiler_params=pltpu.CompilerParams(needs_layout_passes=False),
    )
    def k(pos_hbm, cnt_hbm, out, prow0, prow1, prow2, prow3, cnt_v, obuf,
          *rest):
        prows = (prow0, prow1, prow2, prow3)
        rbufs = rest[:nq]
        sins = rest[nq:nq + 4]
        wid = lax.axis_index("s") * 2 + lax.axis_index("c")
        b = wid // 2
        s0 = (wid % 2) * SB
        bn = b * N
        l16 = lax.broadcasted_iota(jnp.int32, (16,), 0)
        z16 = jnp.zeros((16,), jnp.int32)
        pltpu.sync_copy(cnt_hbm.at[b, pl.ds(s0, SB)], cnt_v)
        for w in range(4):
            pltpu.async_copy(pos_hbm.at[b, s0 + w], prows[w], sins[w])

        def process(i, prow):
            for q in range(nq):
                rbufs[q][pl.ds(0, 16)] = z16 + bn

            def vec_body(u, _c):
                for w in range(4):
                    v = u * 4 + w
                    pv = prow[pl.ds(v * 16, 16)]
                    idxv = l16 + (bn + v * 16)
                    for q in range(nq):
                        pe = (pv >> (8 * q)) & 255
                        plsc.store_scatter(rbufs[q], [pe], idxv)
                return 0

            lax.fori_loop(0, N // 64, vec_body, 0)
            iv = z16 + i
            for q in range(nq):
                cql = plsc.load_gather(cnt_v, [iv, z16])
                cq = (cql >> (8 * q)) & 255
                for j in range(ks[q] // 16):
                    posj = l16 + j * 16
                    gidx = jnp.where(posj < cq, posj, 0)
                    vec = plsc.load_gather(rbufs[q], [gidx])
                    obuf[pl.ds(koff[q] + j * 16, 16)] = vec
            pltpu.sync_copy(obuf, out.at[b, s0 + i])

        def row_body(j, _):
            i0 = 4 * j
            for w in range(4):
                pltpu.make_async_copy(pos_hbm.at[b, s0], prows[w],
                                      sins[w]).wait()
                process(i0 + w, prows[w])
                pltpu.async_copy(
                    pos_hbm.at[b, s0 + jnp.minimum(i0 + w + 4, SB - 1)],
                    prows[w], sins[w])
            return 0

        lax.fori_loop(0, SB // 4, row_body, 0)
        for w in range(4):
            pltpu.make_async_copy(pos_hbm.at[b, s0], prows[w], sins[w]).wait()

    return k


def _make_gather(rows, dtype=jnp.float32):
    per_w = rows // NW
    ch = 256 if per_w % 256 == 0 else 128
    n_ch = per_w // ch
    assert n_ch % 2 == 0 or n_ch == 1
    mesh = plsc.VectorSubcoreMesh(core_axis_name="c", subcore_axis_name="s")

    @functools.partial(
        pl.kernel,
        out_type=jax.ShapeDtypeStruct((rows, DP), dtype),
        mesh=mesh,
        scratch_types=[
            pltpu.VMEM((ch,), jnp.int32), pltpu.VMEM((ch,), jnp.int32),
            pltpu.VMEM((ch, DP), dtype), pltpu.VMEM((ch, DP), dtype),
            pltpu.SemaphoreType.DMA, pltpu.SemaphoreType.DMA,
            pltpu.SemaphoreType.DMA, pltpu.SemaphoreType.DMA,
        ],
    )
    def k(tab_hbm, idx_hbm, out_hbm, idx_a, idx_b, rows_a, rows_b,
          sga, sgb, soa, sob):
        wid = lax.axis_index("s") * 2 + lax.axis_index("c")
        base_w = wid * per_w

        def fire(c, idx_v, rows_v, sem):
            pltpu.sync_copy(idx_hbm.at[pl.ds(base_w + c * ch, ch)], idx_v)
            for j in range(ch // 128):
                pltpu.async_copy(
                    tab_hbm.at[idx_v.at[pl.ds(j * 128, 128)]],
                    rows_v.at[pl.ds(j * 128, 128)], sem)

        def gwait(idx_v, rows_v, sem):
            for j in range(ch // 128):
                pltpu.make_async_copy(
                    tab_hbm.at[idx_v.at[pl.ds(j * 128, 128)]],
                    rows_v.at[pl.ds(j * 128, 128)], sem).wait()

        def owait(rows_v, c, sem):
            pltpu.make_async_copy(
                rows_v, out_hbm.at[pl.ds(base_w, ch)], sem).wait()

        if n_ch == 1:
            fire(0, idx_a, rows_a, sga)
            gwait(idx_a, rows_a, sga)
            pltpu.sync_copy(rows_a, out_hbm.at[pl.ds(base_w, ch)])
            return

        fire(0, idx_a, rows_a, sga)
        fire(1, idx_b, rows_b, sgb)

        def body(h, _):
            c0 = 2 * h
            gwait(idx_a, rows_a, sga)
            pltpu.async_copy(rows_a, out_hbm.at[pl.ds(base_w + c0 * ch, ch)],
                             soa)
            gwait(idx_b, rows_b, sgb)
            pltpu.async_copy(
                rows_b, out_hbm.at[pl.ds(base_w + (c0 + 1) * ch, ch)], sob)
            nx0 = jnp.minimum(c0 + 2, n_ch - 2)
            nx1 = jnp.minimum(c0 + 3, n_ch - 1)
            owait(rows_a, c0, soa)
            fire(nx0, idx_a, rows_a, sga)
            owait(rows_b, c0 + 1, sob)
            fire(nx1, idx_b, rows_b, sgb)
            return 0

        lax.fori_loop(0, n_ch // 2, body, 0)
        gwait(idx_a, rows_a, sga)
        gwait(idx_b, rows_b, sgb)

    return k


def _prep_layers(params, pad_first_to=DP):
    out = []
    for i, (W, b, gamma, beta) in enumerate(params):
        Wt = jnp.transpose(W)
        if i == 0 and Wt.shape[0] < pad_first_to:
            Wt = jnp.concatenate(
                [Wt, jnp.zeros((pad_first_to - Wt.shape[0], Wt.shape[1]),
                               jnp.float32)], axis=0)
        s = gamma / jnp.sqrt(1.0 + EPS)
        out.append((Wt, b[None, :], s[None, :], beta[None, :]))
    return out


def _mlp(x, wrefs):
    for (w, b, s, t) in wrefs:
        y = jnp.dot(x, w[...])
        y = (y + b[...]) * s[...] + t[...]
        x = jnp.maximum(y, 0.0)
    return x


def _adj_center(x3, cen, cpb, kk):
    li4 = lax.broadcasted_iota(jnp.int32, (cpb, 4), 1)
    li = lax.broadcasted_iota(jnp.int32, (cpb, kk, DP), 2)
    adj = jnp.zeros((cpb, kk, DP), jnp.float32)
    for d in range(3):
        cd = jnp.sum(jnp.where(li4 == d, cen, 0.0), axis=1, keepdims=True)
        adj = adj + jnp.where(li == 64 + d, cd[:, :, None], 0.0)
    return x3 - adj


_RCPB = 64


def _refine_body(nl, g_ref, c_ref, *rest):
    wrefs = [tuple(rest[4 * i:4 * i + 4]) for i in range(nl)]
    out_ref = rest[4 * nl]
    x3 = g_ref[...]
    cen = c_ref[...]
    xf = _adj_center(x3, cen, _RCPB, 16)
    x2 = xf.reshape(_RCPB * 16, DP)
    h = _mlp(x2, wrefs)
    h3 = h.reshape(_RCPB, 16, 1)
    m = h3
    for hh in (8, 4, 2, 1):
        m = jnp.maximum(m[:, :hh], m[:, hh:2 * hh])
    e = jnp.exp(h3 - m)
    ssum = e
    for hh in (8, 4, 2, 1):
        ssum = ssum[:, :hh] + ssum[:, hh:2 * hh]
    p = e / ssum
    w = p * x3
    for hh in (8, 4, 2, 1):
        w = w[:, :hh] + w[:, hh:2 * hh]
    out_ref[...] = w.reshape(_RCPB, DP)


def _refine(g0, cen0, layers):
    nl = len(layers)
    flat = [a for l in layers for a in l]
    wspecs = [pl.BlockSpec(a.shape, lambda i: tuple(0 for _ in a.shape))
              for a in flat]
    return pl.pallas_call(
        functools.partial(_refine_body, nl),
        grid=(B * S // _RCPB,),
        in_specs=[
            pl.BlockSpec((_RCPB, 16, DP), lambda i: (i, 0, 0)),
            pl.BlockSpec((_RCPB, 4), lambda i: (i, 0)),
        ] + wspecs,
        out_specs=pl.BlockSpec((_RCPB, DP), lambda i: (i, 0)),
        out_shape=jax.ShapeDtypeStruct((B * S, DP), jnp.float32),
    )(g0, cen0, *flat)


def _branch_body(nl, kk, cpb, g_ref, c_ref, *rest):
    wrefs = [tuple(rest[4 * i:4 * i + 4]) for i in range(nl)]
    out_ref = rest[4 * nl]
    x3 = g_ref[...]
    cen = c_ref[...]
    xf = _adj_center(x3, cen, cpb, kk)
    x2 = xf.reshape(cpb * kk, DP)
    y = _mlp(x2, wrefs)
    c_out = y.shape[1]
    y3 = y.reshape(cpb, kk, c_out)
    hh = kk // 2
    while hh >= 1:
        y3 = jnp.maximum(y3[:, :hh], y3[:, hh:2 * hh])
        hh //= 2
    out_ref[...] = y3.reshape(cpb, c_out)


def _branch(g, cen, layers, kk):
    nl = len(layers)
    c_out = layers[-1][0].shape[1]
    cpb = max(1, 1024 // kk)
    flat = [a for l in layers for a in l]
    wspecs = [pl.BlockSpec(a.shape, lambda i: tuple(0 for _ in a.shape))
              for a in flat]
    return pl.pallas_call(
        functools.partial(_branch_body, nl, kk, cpb),
        grid=(B * S // cpb,),
        in_specs=[
            pl.BlockSpec((cpb, kk, DP), lambda i: (i, 0, 0)),
            pl.BlockSpec((cpb, 4), lambda i: (i, 0)),
        ] + wspecs,
        out_specs=pl.BlockSpec((cpb, c_out), lambda i: (i, 0)),
        out_shape=jax.ShapeDtypeStruct((B * S, c_out), jnp.float32),
    )(g, cen, *flat)


def _ballq_jax_TEMP(specs):
    def f(pos, cnt):
        outs = []
        for q, (r2, kq) in enumerate(specs):
            pq = (pos >> (8 * q)) & 255
            vals = jnp.where(pq < kq, pq, 255)
            idxs = jnp.argsort(vals, axis=-1, stable=True
                               ).astype(jnp.int32)[:, :, :kq]
            cq = (cnt[:, :, 0] >> (8 * q)) & 255
            kpos = jnp.arange(kq, dtype=jnp.int32)[None, None, :]
            first = idxs[:, :, :1]
            out = jnp.where(kpos < cq[:, :, None], idxs, first)
            out = jnp.where(cq[:, :, None] == 0, 0, out)
            out = out + jnp.arange(B, dtype=jnp.int32)[:, None, None] * N
            outs.append(out)
        return tuple(outs)
    return f


def kernel(xyz, points, refine_params, msg_params):
    x = xyz[:, 0, :]
    y = xyz[:, 1, :]
    z = xyz[:, 2, :]
    far0 = jax.random.randint(jax.random.key(1), (B,), 0, N,
                              dtype=jnp.int32).reshape(B, 1)
    ox, oy, oz = _fps(x, y, z, far0)
    cen0 = jnp.stack([ox, oy, oz, jnp.zeros_like(ox)], axis=-1)
    p4 = jnp.concatenate([xyz, jnp.zeros((B, 1, N), jnp.float32)], axis=1)
    spec0 = ((RADII[0] ** 2, 16),)
    pos0, cnt0 = _dist_rank(cen0, p4, spec0)
    g0 = _make_ballq(spec0)(pos0, cnt0)

    tab = jnp.concatenate(
        [jnp.transpose(points, (0, 2, 1)), jnp.transpose(xyz, (0, 2, 1)),
         jnp.zeros((B, N, DP - CIN - 3), jnp.float32)],
        axis=-1).reshape(B * N, DP)

    G0 = _make_gather(B * S * 16)(tab, g0.reshape(-1))
    rlayers = _prep_layers(refine_params)
    cen1_dp = _refine(G0.reshape(B * S, 16, DP), cen0.reshape(B * S, 4),
                      rlayers)
    cen1 = cen1_dp[:, 64:67]
    cen1_4 = jnp.concatenate(
        [cen1, jnp.zeros((B * S, 1), jnp.float32)], axis=1)

    spec1 = tuple((r ** 2, k) for r, k in zip(RADII, KS))
    pos1, cnt1 = _dist_rank(cen1_4.reshape(B, S, 4), p4, spec1)
    gall = _make_ballq(spec1)(pos1, cnt1)
    gs = (gall[:, :, :16], gall[:, :, 16:48], gall[:, :, 48:176])

    outs = []
    for i, kk in enumerate(KS):
        Gi = _make_gather(B * S * kk)(tab, gs[i].reshape(-1))
        blayers = _prep_layers(msg_params[i])
        outs.append(_branch(Gi.reshape(B * S, kk, DP), cen1_4, blayers, kk))

    new_xyz_out = jnp.transpose(cen1.reshape(B, S, 3), (0, 2, 1))
    new_points = jnp.concatenate(
        [jnp.transpose(o.reshape(B, S, -1), (0, 2, 1)) for o in outs], axis=1)
    return new_xyz_out, new_points


# speedup vs baseline: 1.0008x; 1.0003x over previous
import functools

import jax
import jax.numpy as jnp
from jax import lax
from jax.experimental import pallas as pl
from jax.experimental.pallas import tpu as pltpu
from jax.experimental.pallas import tpu_sc as plsc

B = 16
N = 2048
S = 512
CIN = 64
EPS = 1e-5
RADII = (0.1, 0.2, 0.4)
KS = (16, 32, 128)
DP = 128
NW = 32
SB = S // 2
_CH = 128


def _fps_body(x_ref, y_ref, z_ref, f_ref, ox_ref, oy_ref, oz_ref, dist_ref):
    x = x_ref[...]
    y = y_ref[...]
    z = z_ref[...]
    iota = lax.broadcasted_iota(jnp.int32, (B, N), 1)
    iota_s = lax.broadcasted_iota(jnp.int32, (B, S), 1)
    dist_ref[...] = jnp.full((B, N), 1e10, jnp.float32)
    ox_ref[...] = jnp.zeros((B, S), jnp.float32)
    oy_ref[...] = jnp.zeros((B, S), jnp.float32)
    oz_ref[...] = jnp.zeros((B, S), jnp.float32)

    def body(s, nidx):
        mf = (iota == nidx).astype(jnp.float32)
        cx = jnp.sum(x * mf, axis=1, keepdims=True)
        cy = jnp.sum(y * mf, axis=1, keepdims=True)
        cz = jnp.sum(z * mf, axis=1, keepdims=True)
        sel = iota_s == s
        ox_ref[...] = jnp.where(sel, cx, ox_ref[...])
        oy_ref[...] = jnp.where(sel, cy, oy_ref[...])
        oz_ref[...] = jnp.where(sel, cz, oz_ref[...])
        dx = x - cx
        dy = y - cy
        dz = z - cz
        d = dx * dx + dy * dy
        d = d + dz * dz
        dist = jnp.minimum(dist_ref[...], d)
        dist_ref[...] = dist
        mx = jnp.max(dist, axis=1, keepdims=True)
        return jnp.min(jnp.where(dist == mx, iota, N), axis=1, keepdims=True)

    lax.fori_loop(0, S, body, f_ref[...])


def _fps(x, y, z, far0):
    out = jax.ShapeDtypeStruct((B, S), jnp.float32)
    return pl.pallas_call(
        _fps_body,
        out_shape=(out, out, out),
        scratch_shapes=[pltpu.VMEM((B, N), jnp.float32)],
    )(x, y, z, far0)


_SBD = 128


def _dist_rank_body(specs, c_ref, p_ref, pos_ref, cnt_ref):
    a = c_ref[0]
    pb = p_ref[0]
    mm = jnp.dot(a, pb)
    ssrc = jnp.sum(a * a, axis=1, keepdims=True)
    sdst = jnp.sum(pb * pb, axis=0, keepdims=True)
    d = -2.0 * mm
    d = d + ssrc
    d = d + sdst
    ii = lax.broadcasted_iota(jnp.int32, (_CH, _CH), 0)
    jj = lax.broadcasted_iota(jnp.int32, (_CH, _CH), 1)
    tri = (ii <= jj).astype(jnp.float32)
    lane = lax.broadcasted_iota(jnp.int32, (_SBD, _CH), 1)
    bases = [jnp.zeros((_SBD, 1), jnp.float32) for _ in specs]
    for c in range(N // _CH):
        dc = d[:, c * _CH:(c + 1) * _CH]
        pk = jnp.zeros((_SBD, _CH), jnp.int32)
        for q, (r2, kq) in enumerate(specs):
            mc = (dc <= r2).astype(jnp.float32)
            pref = jnp.dot(mc, tri)
            rank = pref + bases[q] - 1.0
            valid = (mc > 0.0) & (rank < kq)
            dump = jnp.asarray(float(kq), jnp.float32) + (lane % 16).astype(
                jnp.float32)
            pq = jnp.where(valid, rank, dump).astype(jnp.int32)
            pk = pk + (pq << (8 * q))
            last = jnp.sum(jnp.where(lane == _CH - 1, pref, 0.0),
                           axis=1, keepdims=True)
            bases[q] = bases[q] + last
        pos_ref[0, :, c * _CH:(c + 1) * _CH] = pk
    cnt = jnp.zeros((_SBD, 1), jnp.int32)
    for q, (r2, kq) in enumerate(specs):
        cq = jnp.minimum(bases[q], float(kq)).astype(jnp.int32)
        cnt = cnt + (cq << (8 * q))
    cnt_ref[0, :, :] = jnp.broadcast_to(cnt, (_SBD, 8))


def _dist_rank(cen4, p4, specs):
    return pl.pallas_call(
        functools.partial(_dist_rank_body, specs),
        grid=(B, S // _SBD),
        in_specs=[
            pl.BlockSpec((1, _SBD, 4), lambda b, i: (b, i, 0)),
            pl.BlockSpec((1, 4, N), lambda b, i: (b, 0, 0)),
        ],
        out_specs=[
            pl.BlockSpec((1, _SBD, N), lambda b, i: (b, i, 0)),
            pl.BlockSpec((1, _SBD, 8), lambda b, i: (b, i, 0)),
        ],
        out_shape=[
            jax.ShapeDtypeStruct((B, S, N), jnp.int32),
            jax.ShapeDtypeStruct((B, S, 8), jnp.int32),
        ],
    )(cen4, p4)


def _make_ballq(specs):
    nq = len(specs)
    ks = [k for _, k in specs]
    sumk = sum(ks)
    koff = [sum(ks[:q]) for q in range(nq)]
    mesh = plsc.VectorSubcoreMesh(core_axis_name="c", subcore_axis_name="s")
    scratch = [pltpu.VMEM((N,), jnp.int32) for _ in range(4)]
    scratch += [pltpu.VMEM((SB, 8), jnp.int32), pltpu.VMEM((sumk,), jnp.int32)]
    scratch += [pltpu.VMEM((k + 16,), jnp.int32) for k in ks]
    scratch += [pltpu.SemaphoreType.DMA for _ in range(4)]

    @functools.partial(
        pl.kernel,
        out_type=jax.ShapeDtypeStruct((B, S, sumk), jnp.int32),
        mesh=mesh,
        scratch_types=scratch,
        compiler_params=pltpu.CompilerParams(needs_layout_passes=False),
    )
    def k(pos_hbm, cnt_hbm, out, prow0, prow1, prow2, prow3, cnt_v, obuf,
          *rest):
        prows = (prow0, prow1, prow2, prow3)
        rbufs = rest[:nq]
        sins = rest[nq:nq + 4]
        wid = lax.axis_index("s") * 2 + lax.axis_index("c")
        b = wid // 2
        s0 = (wid % 2) * SB
        bn = b * N
        l16 = lax.broadcasted_iota(jnp.int32, (16,), 0)
        z16 = jnp.zeros((16,), jnp.int32)
        pltpu.sync_copy(cnt_hbm.at[b, pl.ds(s0, SB)], cnt_v)
        for w in range(4):
            pltpu.async_copy(pos_hbm.at[b, s0 + w], prows[w], sins[w])

        def process(i, prow):
            for q in range(nq):
                rbufs[q][pl.ds(0, 16)] = z16 + bn

            def vec_body(u, _c):
                for w in range(4):
                    v = u * 4 + w
                    pv = prow[pl.ds(v * 16, 16)]
                    idxv = l16 + (bn + v * 16)
                    for q in range(nq):
                        pe = (pv >> (8 * q)) & 255
                        plsc.store_scatter(rbufs[q], [pe], idxv)
                return 0

            lax.fori_loop(0, N // 64, vec_body, 0)
            iv = z16 + i
            for q in range(nq):
                cql = plsc.load_gather(cnt_v, [iv, z16])
                cq = (cql >> (8 * q)) & 255
                for j in range(ks[q] // 16):
                    posj = l16 + j * 16
                    gidx = jnp.where(posj < cq, posj, 0)
                    vec = plsc.load_gather(rbufs[q], [gidx])
                    obuf[pl.ds(koff[q] + j * 16, 16)] = vec
            pltpu.sync_copy(obuf, out.at[b, s0 + i])

        def row_body(j, _):
            i0 = 4 * j
            for w in range(4):
                pltpu.make_async_copy(pos_hbm.at[b, s0], prows[w],
                                      sins[w]).wait()
                process(i0 + w, prows[w])
                pltpu.async_copy(
                    pos_hbm.at[b, s0 + jnp.minimum(i0 + w + 4, SB - 1)],
                    prows[w], sins[w])
            return 0

        lax.fori_loop(0, SB // 4, row_body, 0)
        for w in range(4):
            pltpu.make_async_copy(pos_hbm.at[b, s0], prows[w], sins[w]).wait()

    return k


def _make_gather(rows, dtype=jnp.float32):
    per_w = rows // NW
    ch = 256 if per_w % 256 == 0 else 128
    n_ch = per_w // ch
    assert n_ch % 2 == 0 or n_ch == 1
    mesh = plsc.VectorSubcoreMesh(core_axis_name="c", subcore_axis_name="s")

    @functools.partial(
        pl.kernel,
        out_type=jax.ShapeDtypeStruct((rows, DP), dtype),
        mesh=mesh,
        scratch_types=[
            pltpu.VMEM((ch,), jnp.int32), pltpu.VMEM((ch,), jnp.int32),
            pltpu.VMEM((ch, DP), dtype), pltpu.VMEM((ch, DP), dtype),
            pltpu.SemaphoreType.DMA, pltpu.SemaphoreType.DMA,
            pltpu.SemaphoreType.DMA, pltpu.SemaphoreType.DMA,
        ],
    )
    def k(tab_hbm, idx_hbm, out_hbm, idx_a, idx_b, rows_a, rows_b,
          sga, sgb, soa, sob):
        wid = lax.axis_index("s") * 2 + lax.axis_index("c")
        base_w = wid * per_w

        def fire(c, idx_v, rows_v, sem):
            pltpu.sync_copy(idx_hbm.at[pl.ds(base_w + c * ch, ch)], idx_v)
            for j in range(ch // 128):
                pltpu.async_copy(
                    tab_hbm.at[idx_v.at[pl.ds(j * 128, 128)]],
                    rows_v.at[pl.ds(j * 128, 128)], sem)

        def gwait(idx_v, rows_v, sem):
            for j in range(ch // 128):
                pltpu.make_async_copy(
                    tab_hbm.at[idx_v.at[pl.ds(j * 128, 128)]],
                    rows_v.at[pl.ds(j * 128, 128)], sem).wait()

        def owait(rows_v, c, sem):
            pltpu.make_async_copy(
                rows_v, out_hbm.at[pl.ds(base_w, ch)], sem).wait()

        if n_ch == 1:
            fire(0, idx_a, rows_a, sga)
            gwait(idx_a, rows_a, sga)
            pltpu.sync_copy(rows_a, out_hbm.at[pl.ds(base_w, ch)])
            return

        fire(0, idx_a, rows_a, sga)
        fire(1, idx_b, rows_b, sgb)

        def body(h, _):
            c0 = 2 * h
            gwait(idx_a, rows_a, sga)
            pltpu.async_copy(rows_a, out_hbm.at[pl.ds(base_w + c0 * ch, ch)],
                             soa)
            gwait(idx_b, rows_b, sgb)
            pltpu.async_copy(
                rows_b, out_hbm.at[pl.ds(base_w + (c0 + 1) * ch, ch)], sob)
            nx0 = jnp.minimum(c0 + 2, n_ch - 2)
            nx1 = jnp.minimum(c0 + 3, n_ch - 1)
            owait(rows_a, c0, soa)
            fire(nx0, idx_a, rows_a, sga)
            owait(rows_b, c0 + 1, sob)
            fire(nx1, idx_b, rows_b, sgb)
            return 0

        lax.fori_loop(0, n_ch // 2, body, 0)
        gwait(idx_a, rows_a, sga)
        gwait(idx_b, rows_b, sgb)

    return k


def _prep_layers(params, pad_first_to=DP):
    out = []
    for i, (W, b, gamma, beta) in enumerate(params):
        Wt = jnp.transpose(W)
        if i == 0 and Wt.shape[0] < pad_first_to:
            Wt = jnp.concatenate(
                [Wt, jnp.zeros((pad_first_to - Wt.shape[0], Wt.shape[1]),
                               jnp.float32)], axis=0)
        s = gamma / jnp.sqrt(1.0 + EPS)
        out.append((Wt, b[None, :], s[None, :], beta[None, :]))
    return out


def _mlp(x, wrefs):
    for (w, b, s, t) in wrefs:
        y = jnp.dot(x, w[...])
        y = (y + b[...]) * s[...] + t[...]
        x = jnp.maximum(y, 0.0)
    return x


def _adj_center(x3, cen, cpb, kk):
    li4 = lax.broadcasted_iota(jnp.int32, (cpb, 4), 1)
    li = lax.broadcasted_iota(jnp.int32, (cpb, kk, DP), 2)
    adj = jnp.zeros((cpb, kk, DP), jnp.float32)
    for d in range(3):
        cd = jnp.sum(jnp.where(li4 == d, cen, 0.0), axis=1, keepdims=True)
        adj = adj + jnp.where(li == 64 + d, cd[:, :, None], 0.0)
    return x3 - adj


_RCPB = 64


def _refine_body(nl, g_ref, c_ref, *rest):
    wrefs = [tuple(rest[4 * i:4 * i + 4]) for i in range(nl)]
    out_ref = rest[4 * nl]
    x3 = g_ref[...]
    cen = c_ref[...]
    xf = _adj_center(x3, cen, _RCPB, 16)
    x2 = xf.reshape(_RCPB * 16, DP)
    h = _mlp(x2, wrefs)
    h3 = h.reshape(_RCPB, 16, 1)
    m = h3
    for hh in (8, 4, 2, 1):
        m = jnp.maximum(m[:, :hh], m[:, hh:2 * hh])
    e = jnp.exp(h3 - m)
    ssum = e
    for hh in (8, 4, 2, 1):
        ssum = ssum[:, :hh] + ssum[:, hh:2 * hh]
    p = e / ssum
    w = p * x3
    for hh in (8, 4, 2, 1):
        w = w[:, :hh] + w[:, hh:2 * hh]
    out_ref[...] = w.reshape(_RCPB, DP)


def _refine(g0, cen0, layers):
    nl = len(layers)
    flat = [a for l in layers for a in l]
    wspecs = [pl.BlockSpec(a.shape, lambda i: tuple(0 for _ in a.shape))
              for a in flat]
    return pl.pallas_call(
        functools.partial(_refine_body, nl),
        grid=(B * S // _RCPB,),
        in_specs=[
            pl.BlockSpec((_RCPB, 16, DP), lambda i: (i, 0, 0)),
            pl.BlockSpec((_RCPB, 4), lambda i: (i, 0)),
        ] + wspecs,
        out_specs=pl.BlockSpec((_RCPB, DP), lambda i: (i, 0)),
        out_shape=jax.ShapeDtypeStruct((B * S, DP), jnp.float32),
    )(g0, cen0, *flat)


def _branch_body(nl, kk, cpb, g_ref, c_ref, *rest):
    wrefs = [tuple(rest[4 * i:4 * i + 4]) for i in range(nl)]
    out_ref = rest[4 * nl]
    x3 = g_ref[...]
    cen = c_ref[...]
    xf = _adj_center(x3, cen, cpb, kk)
    x2 = xf.reshape(cpb * kk, DP)
    y = _mlp(x2, wrefs)
    c_out = y.shape[1]
    y3 = y.reshape(cpb, kk, c_out)
    hh = kk // 2
    while hh >= 1:
        y3 = jnp.maximum(y3[:, :hh], y3[:, hh:2 * hh])
        hh //= 2
    out_ref[...] = y3.reshape(cpb, c_out)


def _branch(g, cen, layers, kk):
    nl = len(layers)
    c_out = layers[-1][0].shape[1]
    cpb = max(1, 1024 // kk)
    flat = [a for l in layers for a in l]
    wspecs = [pl.BlockSpec(a.shape, lambda i: tuple(0 for _ in a.shape))
              for a in flat]
    return pl.pallas_call(
        functools.partial(_branch_body, nl, kk, cpb),
        grid=(B * S // cpb,),
        in_specs=[
            pl.BlockSpec((cpb, kk, DP), lambda i: (i, 0, 0)),
            pl.BlockSpec((cpb, 4), lambda i: (i, 0)),
        ] + wspecs,
        out_specs=pl.BlockSpec((cpb, c_out), lambda i: (i, 0)),
        out_shape=jax.ShapeDtypeStruct((B * S, c_out), jnp.float32),
    )(g, cen, *flat)


def kernel(xyz, points, refine_params, msg_params):
    x = xyz[:, 0, :]
    y = xyz[:, 1, :]
    z = xyz[:, 2, :]
    far0 = jax.random.randint(jax.random.key(1), (B,), 0, N,
                              dtype=jnp.int32).reshape(B, 1)
    ox, oy, oz = _fps(x, y, z, far0)
    cen0 = jnp.stack([ox, oy, oz, jnp.zeros_like(ox)], axis=-1)
    p4 = jnp.concatenate([xyz, jnp.zeros((B, 1, N), jnp.float32)], axis=1)
    spec0 = ((RADII[0] ** 2, 16),)
    pos0, cnt0 = _dist_rank(cen0, p4, spec0)
    g0 = _make_ballq(spec0)(pos0, cnt0)

    tab = jnp.concatenate(
        [jnp.transpose(points, (0, 2, 1)), jnp.transpose(xyz, (0, 2, 1)),
         jnp.zeros((B, N, DP - CIN - 3), jnp.float32)],
        axis=-1).reshape(B * N, DP)

    G0 = _make_gather(B * S * 16)(tab, g0.reshape(-1))
    rlayers = _prep_layers(refine_params)
    cen1_dp = _refine(G0.reshape(B * S, 16, DP), cen0.reshape(B * S, 4),
                      rlayers)
    cen1 = cen1_dp[:, 64:67]
    cen1_4 = jnp.concatenate(
        [cen1, jnp.zeros((B * S, 1), jnp.float32)], axis=1)

    spec1 = tuple((r ** 2, k) for r, k in zip(RADII, KS))
    pos1, cnt1 = _dist_rank(cen1_4.reshape(B, S, 4), p4, spec1)
    gall = _make_ballq(spec1)(pos1, cnt1)
    gs = (gall[:, :, :16], gall[:, :, 16:48], gall[:, :, 48:176])

    outs = []
    for i, kk in enumerate(KS):
        Gi = _make_gather(B * S * kk)(tab, gs[i].reshape(-1))
        blayers = _prep_layers(msg_params[i])
        outs.append(_branch(Gi.reshape(B * S, kk, DP), cen1_4, blayers, kk))

    new_xyz_out = jnp.transpose(cen1.reshape(B, S, 3), (0, 2, 1))
    new_points = jnp.concatenate(
        [jnp.transpose(o.reshape(B, S, -1), (0, 2, 1)) for o in outs], axis=1)
    return new_xyz_out, new_points
